# baseline (speedup 1.0000x reference)
import jax
import jax.numpy as jnp
from jax.experimental import pallas as pl
from jax.experimental.pallas import tpu as pltpu

_BLOCK = 2048
_N = 10000
_TAIL = _N - (_N // _BLOCK) * _BLOCK
_NBLK = (_N + _BLOCK - 1) // _BLOCK


def _gates_kernel(x_hbm, wi_ref, wc_ref, wo_ref, bxi_ref, bhi_ref, bi_ref,
                  bxc_ref, bhc_ref, bc_ref, bxo_ref, bho_ref, bo_ref,
                  wco_ref, wlin_ref, blin_ref, o_ref, xbuf, sem):
    step = pl.program_id(0)

    def start_copy(idx):
        slot = jax.lax.rem(idx, 2)

        @pl.when(idx < _NBLK - 1)
        def _():
            pltpu.make_async_copy(
                x_hbm.at[pl.ds(idx * _BLOCK, _BLOCK), :],
                xbuf.at[slot], sem.at[slot]).start()

        @pl.when(idx == _NBLK - 1)
        def _():
            pltpu.make_async_copy(
                x_hbm.at[pl.ds(idx * _BLOCK, _TAIL), :],
                xbuf.at[slot, 0:_TAIL], sem.at[slot]).start()

    @pl.when(step == 0)
    def _():
        start_copy(0)

    @pl.when(step + 1 < _NBLK)
    def _():
        start_copy(step + 1)

    slot = jax.lax.rem(step, 2)

    @pl.when(step < _NBLK - 1)
    def _():
        pltpu.make_async_copy(
            x_hbm.at[pl.ds(step * _BLOCK, _BLOCK), :],
            xbuf.at[slot], sem.at[slot]).wait()

    @pl.when(step == _NBLK - 1)
    def _():
        pltpu.make_async_copy(
            x_hbm.at[pl.ds(step * _BLOCK, _TAIL), :],
            xbuf.at[slot, 0:_TAIL], sem.at[slot]).wait()

    f32 = jnp.float32
    xT = xbuf[slot].T
    zi = jnp.dot(wi_ref[...].T, xT, preferred_element_type=f32)
    zc = jnp.dot(wc_ref[...].T, xT, preferred_element_type=f32)
    zo = jnp.dot(wo_ref[...].T, xT, preferred_element_type=f32)
    bi = ((bxi_ref[...] + bhi_ref[...] + bi_ref[...]) * 0.5).T
    bc = (bxc_ref[...] + bhc_ref[...] + bc_ref[...]).T
    bo = ((bxo_ref[...] + bho_ref[...] + bo_ref[...]) * 0.5).T
    wco = (wco_ref[...] * 0.5).T
    i = jnp.tanh(zi * 0.5 + bi) * 0.5 + 0.5
    t = jnp.tanh(zc + bc)
    c = i * t
    o = jnp.tanh(zo * 0.5 + bo + wco * c) * 0.5 + 0.5
    h = jnp.maximum(o * jnp.tanh(c), 0.0)
    row = jnp.dot(wlin_ref[...], h, preferred_element_type=f32)
    o_ref[...] = row + blin_ref[...]


def kernel(x, edge_index, edge_weight, W_x_i, b_x_i, W_h_i, b_h_i, b_i,
           W_x_f, b_x_f, W_h_f, b_h_f, b_f, W_x_c, b_x_c, W_h_c, b_h_c, b_c,
           W_x_o, b_x_o, W_h_o, b_h_o, b_o, w_c_i, w_c_f, w_c_o, W_lin, b_lin):
    n, f_in = x.shape
    f_out = W_x_i.shape[1]

    r = lambda b: b.reshape(1, f_out)
    full = lambda shape: pl.BlockSpec(shape, lambda i: (0, 0))
    out = pl.pallas_call(
        _gates_kernel,
        grid=(_NBLK,),
        in_specs=[
            pl.BlockSpec(memory_space=pltpu.MemorySpace.HBM),
            full((f_in, f_out)), full((f_in, f_out)), full((f_in, f_out)),
            full((1, f_out)), full((1, f_out)), full((1, f_out)),
            full((1, f_out)), full((1, f_out)), full((1, f_out)),
            full((1, f_out)), full((1, f_out)), full((1, f_out)),
            full((1, f_out)), full((1, f_out)), full((1, 1)),
        ],
        out_specs=pl.BlockSpec((1, _BLOCK), lambda i: (0, i)),
        out_shape=jax.ShapeDtypeStruct((1, n), jnp.float32),
        scratch_shapes=[
            pltpu.MemorySpace.VMEM((2, _BLOCK, 128), jnp.float32),
            pltpu.SemaphoreType.DMA((2,)),
        ],
    )(x, W_x_i, W_x_c, W_x_o,
      r(b_x_i), r(b_h_i), b_i, r(b_x_c), r(b_h_c), b_c,
      r(b_x_o), r(b_h_o), b_o, w_c_o, W_lin.reshape(1, f_out),
      b_lin.reshape(1, 1))
    return out.reshape(n, 1)


# speedup vs baseline: 1.1733x; 1.1733x over previous
import jax
import jax.numpy as jnp
from jax.experimental import pallas as pl
from jax.experimental.pallas import tpu as pltpu


def _gates_kernel(x_ref, wi_ref, wc_ref, wo_ref, bxi_ref, bhi_ref, bi_ref,
                  bxc_ref, bhc_ref, bc_ref, bxo_ref, bho_ref, bo_ref,
                  wco_ref, wlin_ref, blin_ref, o_ref):
    f32 = jnp.float32
    xT = x_ref[...].T
    zi = jnp.dot(wi_ref[...].T, xT, preferred_element_type=f32)
    zc = jnp.dot(wc_ref[...].T, xT, preferred_element_type=f32)
    zo = jnp.dot(wo_ref[...].T, xT, preferred_element_type=f32)
    bi = ((bxi_ref[...] + bhi_ref[...] + bi_ref[...]) * 0.5).T
    bc = (bxc_ref[...] + bhc_ref[...] + bc_ref[...]).T
    bo = ((bxo_ref[...] + bho_ref[...] + bo_ref[...]) * 0.5).T
    wco = (wco_ref[...] * 0.5).T
    i = jnp.tanh(zi * 0.5 + bi) * 0.5 + 0.5
    t = jnp.tanh(zc + bc)
    c = i * t
    o = jnp.tanh(zo * 0.5 + bo + wco * c) * 0.5 + 0.5
    h = jnp.maximum(o * jnp.tanh(c), 0.0)
    row = jnp.dot(wlin_ref[...], h, preferred_element_type=f32)
    o_ref[...] = row + blin_ref[...]


def kernel(x, edge_index, edge_weight, W_x_i, b_x_i, W_h_i, b_h_i, b_i,
           W_x_f, b_x_f, W_h_f, b_h_f, b_f, W_x_c, b_x_c, W_h_c, b_h_c, b_c,
           W_x_o, b_x_o, W_h_o, b_h_o, b_o, w_c_i, w_c_f, w_c_o, W_lin, b_lin):
    n, f_in = x.shape
    f_out = W_x_i.shape[1]

    r = lambda b: b.reshape(1, f_out)
    vmem = pl.BlockSpec(memory_space=pltpu.MemorySpace.VMEM)
    out = pl.pallas_call(
        _gates_kernel,
        in_specs=[vmem] * 16,
        out_specs=vmem,
        out_shape=jax.ShapeDtypeStruct((1, n), jnp.float32),
    )(x, W_x_i, W_x_c, W_x_o,
      r(b_x_i), r(b_h_i), b_i, r(b_x_c), r(b_h_c), b_c,
      r(b_x_o), r(b_h_o), b_o, w_c_o, W_lin.reshape(1, f_out),
      b_lin.reshape(1, 1))
    return out.reshape(n, 1)


# speedup vs baseline: 1.5410x; 1.3134x over previous
import jax
import jax.numpy as jnp
from jax.experimental import pallas as pl
from jax.experimental.pallas import tpu as pltpu


def _gates_kernel(x_ref, w3_ref, bxi_ref, bhi_ref, bi_ref,
                  bxc_ref, bhc_ref, bc_ref, bxo_ref, bho_ref, bo_ref,
                  wco_ref, wlin_ref, blin_ref, o_ref):
    f32 = jnp.float32
    xT = x_ref[...].T
    z3 = jnp.dot(w3_ref[...].T, xT, preferred_element_type=f32)
    zi = z3[0:32]
    zc = z3[32:64]
    zo = z3[64:96]
    bi = ((bxi_ref[...] + bhi_ref[...] + bi_ref[...]) * 0.5).T
    bc = (bxc_ref[...] + bhc_ref[...] + bc_ref[...]).T
    bo = ((bxo_ref[...] + bho_ref[...] + bo_ref[...]) * 0.5).T
    wco = (wco_ref[...] * 0.5).T
    i = jnp.tanh(zi * 0.5 + bi) * 0.5 + 0.5
    t = jnp.tanh(zc + bc)
    c = i * t
    o = jnp.tanh(zo * 0.5 + bo + wco * c) * 0.5 + 0.5
    h = jnp.maximum(o * jnp.tanh(c), 0.0)
    row = jnp.dot(wlin_ref[...], h, preferred_element_type=f32)
    o_ref[...] = row + blin_ref[...]


def kernel(x, edge_index, edge_weight, W_x_i, b_x_i, W_h_i, b_h_i, b_i,
           W_x_f, b_x_f, W_h_f, b_h_f, b_f, W_x_c, b_x_c, W_h_c, b_h_c, b_c,
           W_x_o, b_x_o, W_h_o, b_h_o, b_o, w_c_i, w_c_f, w_c_o, W_lin, b_lin):
    n, f_in = x.shape
    f_out = W_x_i.shape[1]

    W3 = jnp.concatenate([W_x_i, W_x_c, W_x_o], axis=1)
    r = lambda b: b.reshape(1, f_out)
    vmem = pl.BlockSpec(memory_space=pltpu.MemorySpace.VMEM)
    out = pl.pallas_call(
        _gates_kernel,
        in_specs=[vmem] * 14,
        out_specs=vmem,
        out_shape=jax.ShapeDtypeStruct((1, n), jnp.float32),
    )(x, W3,
      r(b_x_i), r(b_h_i), b_i, r(b_x_c), r(b_h_c), b_c,
      r(b_x_o), r(b_h_o), b_o, w_c_o, W_lin.reshape(1, f_out),
      b_lin.reshape(1, 1))
    return out.reshape(n, 1)
